# baseline (speedup 1.0000x reference)
import functools

import jax
import jax.numpy as jnp
from jax import lax
from jax.experimental import pallas as pl
from jax.experimental.pallas import tpu as pltpu
from jax.experimental.pallas import tpu_sc as plsc

H = 4
C = 128
NBLK = 2000
EBLK = 4000

_SC_MESH = plsc.VectorSubcoreMesh(core_axis_name="c", subcore_axis_name="s")
_SC_PARAMS = pltpu.CompilerParams(needs_layout_passes=False)


def _bd(a):
    return (jnp.eye(H, dtype=a.dtype)[:, :, None] * a[:, None, :]).reshape(H, C)


def _dense1_body(xa, xp, Wa, ba, Wp, bp, Aa, Ap, zaT, zpT, Sa, Sp):
    za = lax.dot_general(Wa[...], xa[...], (((0,), (1,)), ((), ())),
                         preferred_element_type=jnp.float32) + ba[...]
    zp = lax.dot_general(Wp[...], xp[...], (((0,), (1,)), ((), ())),
                         preferred_element_type=jnp.float32) + bp[...]
    zaT[...] = za
    zpT[...] = zp
    Sa[...] = lax.dot_general(Aa[...], za, (((1,), (0,)), ((), ())),
                              preferred_element_type=jnp.float32)
    Sp[...] = lax.dot_general(Ap[...], zp, (((1,), (0,)), ((), ())),
                              preferred_element_type=jnp.float32)


def _dense1(xa, xp, Wa, ba, Wp, bp, Aa, Ap):
    n = xa.shape[0]
    return pl.pallas_call(
        _dense1_body,
        out_shape=[
            jax.ShapeDtypeStruct((C, n), jnp.float32),
            jax.ShapeDtypeStruct((C, n), jnp.float32),
            jax.ShapeDtypeStruct((16, n), jnp.float32),
            jax.ShapeDtypeStruct((8, n), jnp.float32),
        ],
    )(xa, xp, Wa, ba, Wp, bp, Aa, Ap)


def _dense2_body(o1, o2, attn, p1T, Wa, ba, Wp, bp, Aa, Ap,
                 za2, zp2, Sa2, Sp2):
    w0 = attn[0, 0]
    w1 = attn[0, 1]
    a1 = o1[...] * w0 + o2[...] * w1
    za = lax.dot_general(Wa[...], a1, (((0,), (0,)), ((), ())),
                         preferred_element_type=jnp.float32) + ba[...]
    zp = lax.dot_general(Wp[...], p1T[...], (((0,), (0,)), ((), ())),
                         preferred_element_type=jnp.float32) + bp[...]
    za2[...] = za
    zp2[...] = zp
    Sa2[...] = lax.dot_general(Aa[...], za, (((1,), (0,)), ((), ())),
                               preferred_element_type=jnp.float32)
    Sp2[...] = lax.dot_general(Ap[...], zp, (((1,), (0,)), ((), ())),
                               preferred_element_type=jnp.float32)


def _dense2(o1, o2, attn, p1T, Wa, ba, Wp, bp, Aa, Ap):
    n = o1.shape[1]
    return pl.pallas_call(
        _dense2_body,
        out_shape=[
            jax.ShapeDtypeStruct((C, n), jnp.float32),
            jax.ShapeDtypeStruct((C, n), jnp.float32),
            jax.ShapeDtypeStruct((16, n), jnp.float32),
            jax.ShapeDtypeStruct((8, n), jnp.float32),
        ],
    )(o1, o2, attn, p1T, Wa, ba, Wp, bp, Aa, Ap)


def _stats_body(o1, o2, kW, kb, q, out):
    k1 = jnp.tanh(lax.dot_general(kW[...], o1[...], (((0,), (0,)), ((), ())),
                                  preferred_element_type=jnp.float32) + kb[...])
    k2 = jnp.tanh(lax.dot_general(kW[...], o2[...], (((0,), (0,)), ((), ())),
                                  preferred_element_type=jnp.float32) + kb[...])
    out[0:1, :] = jnp.sum(k1 * q[...], axis=1).reshape(1, C)
    out[1:2, :] = jnp.sum(k2 * q[...], axis=1).reshape(1, C)


def _stats(o1, o2, kW, kb, q):
    return pl.pallas_call(
        _stats_body,
        out_shape=jax.ShapeDtypeStruct((2, C), jnp.float32),
    )(o1, o2, kW, kb, q)


def _final_body(o1, o2, attn, out):
    w0 = attn[0, 0]
    w1 = attn[0, 1]
    out[...] = lax.transpose(o1[...] * w0 + o2[...] * w1, (1, 0))


def _final(o1, o2, attn):
    n = o1.shape[1]
    return pl.pallas_call(
        _final_body,
        out_shape=jax.ShapeDtypeStruct((n, C), jnp.float32),
    )(o1, o2, attn)


@functools.lru_cache(maxsize=None)
def _make_edge_attn(Ns, Nd, EP):
    Nda = Nd + 16

    @functools.partial(
        pl.kernel,
        mesh=_SC_MESH,
        compiler_params=_SC_PARAMS,
        out_type=jax.ShapeDtypeStruct((C * Nd,), jnp.float32),
        scratch_types=(
            [pltpu.VMEM((Ns,), jnp.float32)] * 5
            + [pltpu.VMEM((Nda,), jnp.float32)] * 6
            + [pltpu.VMEM((EBLK,), jnp.int32)] * 2
        ),
    )
    def edge_attn(srcT, ssrc, sdst, esrc, edst, outT,
                  f0, f1, f2, f3, srow, drow, a0, a1, a2, a3, sacc, bs, bd):
        wid = lax.axis_index("s") * 2 + lax.axis_index("c")
        h = wid // (32 // H)
        base = wid * (C // 32)
        feats = (f0, f1, f2, f3)
        accs = (a0, a1, a2, a3)
        for r in range(4):
            pltpu.sync_copy(srcT.at[pl.ds((base + r) * Ns, Ns)], feats[r])
        pltpu.sync_copy(ssrc.at[pl.ds(h * Ns, Ns)], srow)
        pltpu.sync_copy(sdst.at[pl.ds(h * Nd, Nd)], drow.at[pl.ds(0, Nd)])

        @plsc.parallel_loop(0, Nda // 16, unroll=4)
        def zero_acc(i):
            z = jnp.zeros((16,), jnp.float32)
            for a in accs:
                a[pl.ds(i * 16, 16)] = z
            sacc[pl.ds(i * 16, 16)] = z

        drow[pl.ds(Nd, 16)] = jnp.zeros((16,), jnp.float32)

        def chunk_body(cc, carry):
            pltpu.sync_copy(esrc.at[pl.ds(cc * EBLK, EBLK)], bs)
            pltpu.sync_copy(edst.at[pl.ds(cc * EBLK, EBLK)], bd)

            @plsc.parallel_loop(0, EBLK // 16, unroll=16)
            def grp(g):
                si = bs[pl.ds(g * 16, 16)]
                di = bd[pl.ds(g * 16, 16)]
                av = plsc.load_gather(srow, [si])
                bv = plsc.load_gather(drow, [di])
                t = av + bv
                e = jnp.exp(jnp.maximum(t, 0.2 * t))
                plsc.addupdate_scatter(sacc, [di], e)
                for fr, ar in zip(feats, accs):
                    v = plsc.load_gather(fr, [si])
                    plsc.addupdate_scatter(ar, [di], v * e)

            return carry

        lax.fori_loop(0, EP // EBLK, chunk_body, 0)

        @plsc.parallel_loop(0, Nd // 16, unroll=4)
        def fin(i):
            sl = pl.ds(i * 16, 16)
            sv = sacc[sl] + 1e-16
            for a in accs:
                a[sl] = jnp.maximum(a[sl] / sv, 0.0)
        for r in range(4):
            pltpu.sync_copy(accs[r].at[pl.ds(0, Nd)],
                            outT.at[pl.ds((base + r) * Nd, Nd)])

    return edge_attn


def _edge_attn(srcT, ssrc, sdst, esrc, edst, n_dst):
    Ns = srcT.shape[1]
    E = esrc.shape[0]
    EP = ((E + EBLK - 1) // EBLK) * EBLK
    if EP != E:
        esrc = jnp.pad(esrc, (0, EP - E))
        edst = jnp.pad(edst, (0, EP - E), constant_values=n_dst)
    fn = _make_edge_attn(Ns, n_dst, EP)
    out = fn(srcT.reshape(-1), ssrc.reshape(-1), sdst.reshape(-1), esrc, edst)
    return out.reshape(C, n_dst)


def kernel(x_author, x_paper, edge_index_ap, edge_index_pa, edge_index_aa,
           W_a1, b_a1, W_p1, b_p1, asrc_ap1, adst_ap1, asrc_pa1, adst_pa1,
           asrc_aa1, adst_aa1, kW1, kb1, q1,
           W_a2, b_a2, W_p2, b_p2, asrc_ap2, adst_ap2, asrc_pa2, adst_pa2,
           asrc_aa2, adst_aa2, kW2, kb2, q2):
    na = x_author.shape[0]
    npp = x_paper.shape[0]

    A_a1 = jnp.concatenate(
        [_bd(asrc_ap1), _bd(adst_pa1), _bd(asrc_aa1), _bd(adst_aa1)], axis=0)
    A_p1 = jnp.concatenate([_bd(adst_ap1), _bd(asrc_pa1)], axis=0)
    zaT, zpT, Sa, Sp = _dense1(
        x_author, x_paper, W_a1, b_a1.reshape(C, 1), W_p1, b_p1.reshape(C, 1),
        A_a1, A_p1)

    p1T = _edge_attn(zaT, Sa[0:4], Sp[0:4],
                     edge_index_ap[0], edge_index_ap[1], npp)
    o1T = _edge_attn(zpT, Sp[4:8], Sa[4:8],
                     edge_index_pa[0], edge_index_pa[1], na)
    o2T = _edge_attn(zaT, Sa[8:12], Sa[12:16],
                     edge_index_aa[0], edge_index_aa[1], na)

    ks = _stats(o1T, o2T, kW1, kb1.reshape(C, 1), q1.reshape(C, 1))
    attn1 = jax.nn.softmax(ks.sum(axis=1) / na).reshape(1, 2)

    A_a2 = jnp.concatenate(
        [_bd(adst_pa2), _bd(asrc_aa2), _bd(adst_aa2),
         jnp.zeros((4, C), jnp.float32)], axis=0)
    A_p2 = jnp.concatenate(
        [_bd(asrc_pa2), jnp.zeros((4, C), jnp.float32)], axis=0)
    za2T, zp2T, Sa2, Sp2 = _dense2(
        o1T, o2T, attn1, p1T, W_a2, b_a2.reshape(C, 1),
        W_p2, b_p2.reshape(C, 1), A_a2, A_p2)

    o1T2 = _edge_attn(zp2T, Sp2[0:4], Sa2[0:4],
                      edge_index_pa[0], edge_index_pa[1], na)
    o2T2 = _edge_attn(za2T, Sa2[4:8], Sa2[8:12],
                      edge_index_aa[0], edge_index_aa[1], na)

    ks2 = _stats(o1T2, o2T2, kW2, kb2.reshape(C, 1), q2.reshape(C, 1))
    attn2 = jax.nn.softmax(ks2.sum(axis=1) / na).reshape(1, 2)
    return _final(o1T2, o2T2, attn2)


# speedup vs baseline: 1.3627x; 1.3627x over previous
import functools

import jax
import jax.numpy as jnp
from jax import lax
from jax.experimental import pallas as pl
from jax.experimental.pallas import tpu as pltpu
from jax.experimental.pallas import tpu_sc as plsc

H = 4
C = 128
NBLK = 2000
EBLK = 2000

_SC_MESH = plsc.VectorSubcoreMesh(core_axis_name="c", subcore_axis_name="s")
_SC_PARAMS = pltpu.CompilerParams(needs_layout_passes=False)


def _bd(a):
    return (jnp.eye(H, dtype=a.dtype)[:, :, None] * a[:, None, :]).reshape(H, C)


def _dense1_body(xa, xp, Wa, ba, Wp, bp, Aa, Ap, zaT, zpT, Sa, Sp):
    za = lax.dot_general(Wa[...], xa[...], (((0,), (1,)), ((), ())),
                         preferred_element_type=jnp.float32) + ba[...]
    zp = lax.dot_general(Wp[...], xp[...], (((0,), (1,)), ((), ())),
                         preferred_element_type=jnp.float32) + bp[...]
    zaT[...] = za
    zpT[...] = zp
    Sa[...] = lax.dot_general(Aa[...], za, (((1,), (0,)), ((), ())),
                              preferred_element_type=jnp.float32)
    Sp[...] = lax.dot_general(Ap[...], zp, (((1,), (0,)), ((), ())),
                              preferred_element_type=jnp.float32)


def _dense1(xa, xp, Wa, ba, Wp, bp, Aa, Ap):
    n = xa.shape[0]
    return pl.pallas_call(
        _dense1_body,
        out_shape=[
            jax.ShapeDtypeStruct((C, n), jnp.float32),
            jax.ShapeDtypeStruct((C, n), jnp.float32),
            jax.ShapeDtypeStruct((16, n), jnp.float32),
            jax.ShapeDtypeStruct((8, n), jnp.float32),
        ],
    )(xa, xp, Wa, ba, Wp, bp, Aa, Ap)


def _dense2_body(o1, o2, attn, p1T, Wa, ba, Wp, bp, Aa, Ap,
                 za2, zp2, Sa2, Sp2):
    w0 = attn[0, 0]
    w1 = attn[0, 1]
    a1 = o1[...] * w0 + o2[...] * w1
    za = lax.dot_general(Wa[...], a1, (((0,), (0,)), ((), ())),
                         preferred_element_type=jnp.float32) + ba[...]
    zp = lax.dot_general(Wp[...], p1T[...], (((0,), (0,)), ((), ())),
                         preferred_element_type=jnp.float32) + bp[...]
    za2[...] = za
    zp2[...] = zp
    Sa2[...] = lax.dot_general(Aa[...], za, (((1,), (0,)), ((), ())),
                               preferred_element_type=jnp.float32)
    Sp2[...] = lax.dot_general(Ap[...], zp, (((1,), (0,)), ((), ())),
                               preferred_element_type=jnp.float32)


def _dense2(o1, o2, attn, p1T, Wa, ba, Wp, bp, Aa, Ap):
    n = o1.shape[1]
    return pl.pallas_call(
        _dense2_body,
        out_shape=[
            jax.ShapeDtypeStruct((C, n), jnp.float32),
            jax.ShapeDtypeStruct((C, n), jnp.float32),
            jax.ShapeDtypeStruct((16, n), jnp.float32),
            jax.ShapeDtypeStruct((8, n), jnp.float32),
        ],
    )(o1, o2, attn, p1T, Wa, ba, Wp, bp, Aa, Ap)


def _stats_body(o1, o2, kW, kb, q, out):
    k1 = jnp.tanh(lax.dot_general(kW[...], o1[...], (((0,), (0,)), ((), ())),
                                  preferred_element_type=jnp.float32) + kb[...])
    k2 = jnp.tanh(lax.dot_general(kW[...], o2[...], (((0,), (0,)), ((), ())),
                                  preferred_element_type=jnp.float32) + kb[...])
    out[0:1, :] = jnp.sum(k1 * q[...], axis=1).reshape(1, C)
    out[1:2, :] = jnp.sum(k2 * q[...], axis=1).reshape(1, C)


def _stats(o1, o2, kW, kb, q):
    return pl.pallas_call(
        _stats_body,
        out_shape=jax.ShapeDtypeStruct((2, C), jnp.float32),
    )(o1, o2, kW, kb, q)


def _final_body(o1, o2, attn, out):
    w0 = attn[0, 0]
    w1 = attn[0, 1]
    out[...] = lax.transpose(o1[...] * w0 + o2[...] * w1, (1, 0))


def _final(o1, o2, attn):
    n = o1.shape[1]
    return pl.pallas_call(
        _final_body,
        out_shape=jax.ShapeDtypeStruct((n, C), jnp.float32),
    )(o1, o2, attn)


@functools.lru_cache(maxsize=None)
def _make_edge_attn(Ns, Nd, EP):
    Nda = Nd + 16

    @functools.partial(
        pl.kernel,
        mesh=_SC_MESH,
        compiler_params=_SC_PARAMS,
        out_type=jax.ShapeDtypeStruct((C * Nd,), jnp.float32),
        scratch_types=(
            [pltpu.VMEM((Ns,), jnp.float32)] * 5
            + [pltpu.VMEM((Nda,), jnp.float32)] * 6
            + [pltpu.VMEM((EBLK,), jnp.int32)] * 4
            + [pltpu.SemaphoreType.DMA] * 2
        ),
    )
    def edge_attn(srcT, ssrc, sdst, esrc, edst, outT,
                  f0, f1, f2, f3, srow, drow, a0, a1, a2, a3, sacc,
                  bs0, bd0, bs1, bd1, sem0, sem1):
        wid = lax.axis_index("s") * 2 + lax.axis_index("c")
        h = wid // (32 // H)
        base = wid * (C // 32)
        feats = (f0, f1, f2, f3)
        accs = (a0, a1, a2, a3)
        for r in range(4):
            pltpu.sync_copy(srcT.at[pl.ds((base + r) * Ns, Ns)], feats[r])
        pltpu.sync_copy(ssrc.at[pl.ds(h * Ns, Ns)], srow)
        pltpu.sync_copy(sdst.at[pl.ds(h * Nd, Nd)], drow.at[pl.ds(0, Nd)])

        @plsc.parallel_loop(0, Nda // 16, unroll=4)
        def zero_acc(i):
            z = jnp.zeros((16,), jnp.float32)
            for a in accs:
                a[pl.ds(i * 16, 16)] = z
            sacc[pl.ds(i * 16, 16)] = z

        drow[pl.ds(Nd, 16)] = jnp.zeros((16,), jnp.float32)

        nchunks = EP // EBLK
        bufs = ((bs0, bd0, sem0), (bs1, bd1, sem1))

        def issue(cc, buf):
            bs, bd, sem = buf
            pltpu.async_copy(esrc.at[pl.ds(cc * EBLK, EBLK)], bs, sem)
            pltpu.async_copy(edst.at[pl.ds(cc * EBLK, EBLK)], bd, sem)

        def process(buf):
            bs, bd, sem = buf
            pltpu.make_async_copy(esrc.at[pl.ds(0, EBLK)], bs, sem).wait()
            pltpu.make_async_copy(edst.at[pl.ds(0, EBLK)], bd, sem).wait()

            @plsc.parallel_loop(0, EBLK // 16, unroll=8)
            def grp(g):
                si = bs[pl.ds(g * 16, 16)]
                di = bd[pl.ds(g * 16, 16)]
                av = plsc.load_gather(srow, [si])
                bv = plsc.load_gather(drow, [di])
                t = av + bv
                e = jnp.exp(jnp.maximum(t, 0.2 * t))
                plsc.addupdate_scatter(sacc, [di], e)
                for fr, ar in zip(feats, accs):
                    v = plsc.load_gather(fr, [si])
                    plsc.addupdate_scatter(ar, [di], v * e)

        issue(0, bufs[0])
        issue(1, bufs[1])

        def chunk_body(j, carry):
            for b in range(2):
                cc = 2 * j + b
                process(bufs[b])

                @pl.when(cc + 2 < nchunks)
                def _():
                    issue(cc + 2, bufs[b])

            return carry

        lax.fori_loop(0, nchunks // 2, chunk_body, 0)

        @plsc.parallel_loop(0, Nd // 16, unroll=4)
        def fin(i):
            sl = pl.ds(i * 16, 16)
            sv = sacc[sl] + 1e-16
            for a in accs:
                a[sl] = jnp.maximum(a[sl] / sv, 0.0)
        for r in range(4):
            pltpu.sync_copy(accs[r].at[pl.ds(0, Nd)],
                            outT.at[pl.ds((base + r) * Nd, Nd)])

    return edge_attn


def _edge_attn(srcT, ssrc, sdst, esrc, edst, n_dst):
    Ns = srcT.shape[1]
    E = esrc.shape[0]
    EP = ((E + 2 * EBLK - 1) // (2 * EBLK)) * (2 * EBLK)
    if EP != E:
        esrc = jnp.pad(esrc, (0, EP - E))
        edst = jnp.pad(edst, (0, EP - E), constant_values=n_dst)
    fn = _make_edge_attn(Ns, n_dst, EP)
    out = fn(srcT.reshape(-1), ssrc.reshape(-1), sdst.reshape(-1), esrc, edst)
    return out.reshape(C, n_dst)


def kernel(x_author, x_paper, edge_index_ap, edge_index_pa, edge_index_aa,
           W_a1, b_a1, W_p1, b_p1, asrc_ap1, adst_ap1, asrc_pa1, adst_pa1,
           asrc_aa1, adst_aa1, kW1, kb1, q1,
           W_a2, b_a2, W_p2, b_p2, asrc_ap2, adst_ap2, asrc_pa2, adst_pa2,
           asrc_aa2, adst_aa2, kW2, kb2, q2):
    na = x_author.shape[0]
    npp = x_paper.shape[0]

    A_a1 = jnp.concatenate(
        [_bd(asrc_ap1), _bd(adst_pa1), _bd(asrc_aa1), _bd(adst_aa1)], axis=0)
    A_p1 = jnp.concatenate([_bd(adst_ap1), _bd(asrc_pa1)], axis=0)
    zaT, zpT, Sa, Sp = _dense1(
        x_author, x_paper, W_a1, b_a1.reshape(C, 1), W_p1, b_p1.reshape(C, 1),
        A_a1, A_p1)

    p1T = _edge_attn(zaT, Sa[0:4], Sp[0:4],
                     edge_index_ap[0], edge_index_ap[1], npp)
    o1T = _edge_attn(zpT, Sp[4:8], Sa[4:8],
                     edge_index_pa[0], edge_index_pa[1], na)
    o2T = _edge_attn(zaT, Sa[8:12], Sa[12:16],
                     edge_index_aa[0], edge_index_aa[1], na)

    ks = _stats(o1T, o2T, kW1, kb1.reshape(C, 1), q1.reshape(C, 1))
    attn1 = jax.nn.softmax(ks.sum(axis=1) / na).reshape(1, 2)

    A_a2 = jnp.concatenate(
        [_bd(adst_pa2), _bd(asrc_aa2), _bd(adst_aa2),
         jnp.zeros((4, C), jnp.float32)], axis=0)
    A_p2 = jnp.concatenate(
        [_bd(asrc_pa2), jnp.zeros((4, C), jnp.float32)], axis=0)
    za2T, zp2T, Sa2, Sp2 = _dense2(
        o1T, o2T, attn1, p1T, W_a2, b_a2.reshape(C, 1),
        W_p2, b_p2.reshape(C, 1), A_a2, A_p2)

    o1T2 = _edge_attn(zp2T, Sp2[0:4], Sa2[0:4],
                      edge_index_pa[0], edge_index_pa[1], na)
    o2T2 = _edge_attn(za2T, Sa2[4:8], Sa2[8:12],
                      edge_index_aa[0], edge_index_aa[1], na)

    ks2 = _stats(o1T2, o2T2, kW2, kb2.reshape(C, 1), q2.reshape(C, 1))
    attn2 = jax.nn.softmax(ks2.sum(axis=1) / na).reshape(1, 2)
    return _final(o1T2, o2T2, attn2)


# speedup vs baseline: 1.4587x; 1.0704x over previous
import functools

import jax
import jax.numpy as jnp
from jax import lax
from jax.experimental import pallas as pl
from jax.experimental.pallas import tpu as pltpu
from jax.experimental.pallas import tpu_sc as plsc

H = 4
C = 128
NBLK = 2000
EBLK = 4000

_SC_MESH = plsc.VectorSubcoreMesh(core_axis_name="c", subcore_axis_name="s")
_SC_PARAMS = pltpu.CompilerParams(needs_layout_passes=False)


def _bd(a):
    return (jnp.eye(H, dtype=a.dtype)[:, :, None] * a[:, None, :]).reshape(H, C)


def _dense1_body(xa, xp, Wa, ba, Wp, bp, Aa, Ap, zaT, zpT, Sa, Sp):
    za = lax.dot_general(Wa[...], xa[...], (((0,), (1,)), ((), ())),
                         preferred_element_type=jnp.float32) + ba[...]
    zp = lax.dot_general(Wp[...], xp[...], (((0,), (1,)), ((), ())),
                         preferred_element_type=jnp.float32) + bp[...]
    zaT[...] = za
    zpT[...] = zp
    Sa[...] = lax.dot_general(Aa[...], za, (((1,), (0,)), ((), ())),
                              preferred_element_type=jnp.float32)
    Sp[...] = lax.dot_general(Ap[...], zp, (((1,), (0,)), ((), ())),
                              preferred_element_type=jnp.float32)


def _dense1(xa, xp, Wa, ba, Wp, bp, Aa, Ap):
    n = xa.shape[0]
    return pl.pallas_call(
        _dense1_body,
        out_shape=[
            jax.ShapeDtypeStruct((C, n), jnp.float32),
            jax.ShapeDtypeStruct((C, n), jnp.float32),
            jax.ShapeDtypeStruct((16, n), jnp.float32),
            jax.ShapeDtypeStruct((8, n), jnp.float32),
        ],
    )(xa, xp, Wa, ba, Wp, bp, Aa, Ap)


def _dense2_body(o1, o2, attn, p1T, Wa, ba, Wp, bp, Aa, Ap,
                 za2, zp2, Sa2, Sp2):
    w0 = attn[0, 0]
    w1 = attn[0, 1]
    a1 = o1[...] * w0 + o2[...] * w1
    za = lax.dot_general(Wa[...], a1, (((0,), (0,)), ((), ())),
                         preferred_element_type=jnp.float32) + ba[...]
    zp = lax.dot_general(Wp[...], p1T[...], (((0,), (0,)), ((), ())),
                         preferred_element_type=jnp.float32) + bp[...]
    za2[...] = za
    zp2[...] = zp
    Sa2[...] = lax.dot_general(Aa[...], za, (((1,), (0,)), ((), ())),
                               preferred_element_type=jnp.float32)
    Sp2[...] = lax.dot_general(Ap[...], zp, (((1,), (0,)), ((), ())),
                               preferred_element_type=jnp.float32)


def _dense2(o1, o2, attn, p1T, Wa, ba, Wp, bp, Aa, Ap):
    n = o1.shape[1]
    return pl.pallas_call(
        _dense2_body,
        out_shape=[
            jax.ShapeDtypeStruct((C, n), jnp.float32),
            jax.ShapeDtypeStruct((C, n), jnp.float32),
            jax.ShapeDtypeStruct((16, n), jnp.float32),
            jax.ShapeDtypeStruct((8, n), jnp.float32),
        ],
    )(o1, o2, attn, p1T, Wa, ba, Wp, bp, Aa, Ap)


def _stats_body(o1, o2, kW, kb, q, out):
    k1 = jnp.tanh(lax.dot_general(kW[...], o1[...], (((0,), (0,)), ((), ())),
                                  preferred_element_type=jnp.float32) + kb[...])
    k2 = jnp.tanh(lax.dot_general(kW[...], o2[...], (((0,), (0,)), ((), ())),
                                  preferred_element_type=jnp.float32) + kb[...])
    out[0:1, :] = jnp.sum(k1 * q[...], axis=1).reshape(1, C)
    out[1:2, :] = jnp.sum(k2 * q[...], axis=1).reshape(1, C)


def _stats(o1, o2, kW, kb, q):
    return pl.pallas_call(
        _stats_body,
        out_shape=jax.ShapeDtypeStruct((2, C), jnp.float32),
    )(o1, o2, kW, kb, q)


def _final_body(o1, o2, attn, out):
    w0 = attn[0, 0]
    w1 = attn[0, 1]
    out[...] = lax.transpose(o1[...] * w0 + o2[...] * w1, (1, 0))


def _final(o1, o2, attn):
    n = o1.shape[1]
    return pl.pallas_call(
        _final_body,
        out_shape=jax.ShapeDtypeStruct((n, C), jnp.float32),
    )(o1, o2, attn)


@functools.lru_cache(maxsize=None)
def _make_edge_attn(Ns, Nd, EP):
    Nda = Nd + 16

    @functools.partial(
        pl.kernel,
        mesh=_SC_MESH,
        compiler_params=_SC_PARAMS,
        out_type=jax.ShapeDtypeStruct((C * Nd,), jnp.float32),
        scratch_types=(
            [pltpu.VMEM((Ns,), jnp.float32)] * 5
            + [pltpu.VMEM((Nda,), jnp.float32)] * 6
            + [pltpu.VMEM((EBLK,), jnp.int32)] * 4
            + [pltpu.SemaphoreType.DMA] * 2
        ),
    )
    def edge_attn(srcT, ssrc, sdst, esrc, edst, outT,
                  f0, f1, f2, f3, srow, drow, a0, a1, a2, a3, sacc,
                  bs0, bd0, bs1, bd1, sem0, sem1):
        wid = lax.axis_index("s") * 2 + lax.axis_index("c")
        h = wid // (32 // H)
        base = wid * (C // 32)
        feats = (f0, f1, f2, f3)
        accs = (a0, a1, a2, a3)
        for r in range(4):
            pltpu.sync_copy(srcT.at[pl.ds((base + r) * Ns, Ns)], feats[r])
        pltpu.sync_copy(ssrc.at[pl.ds(h * Ns, Ns)], srow)
        pltpu.sync_copy(sdst.at[pl.ds(h * Nd, Nd)], drow.at[pl.ds(0, Nd)])

        @plsc.parallel_loop(0, Nda // 16, unroll=4)
        def zero_acc(i):
            z = jnp.zeros((16,), jnp.float32)
            for a in accs:
                a[pl.ds(i * 16, 16)] = z
            sacc[pl.ds(i * 16, 16)] = z

        drow[pl.ds(Nd, 16)] = jnp.zeros((16,), jnp.float32)

        nchunks = EP // EBLK
        bufs = ((bs0, bd0, sem0), (bs1, bd1, sem1))

        def issue(cc, buf):
            bs, bd, sem = buf
            pltpu.async_copy(esrc.at[pl.ds(cc * EBLK, EBLK)], bs, sem)
            pltpu.async_copy(edst.at[pl.ds(cc * EBLK, EBLK)], bd, sem)

        def process(buf):
            bs, bd, sem = buf
            pltpu.make_async_copy(esrc.at[pl.ds(0, EBLK)], bs, sem).wait()
            pltpu.make_async_copy(edst.at[pl.ds(0, EBLK)], bd, sem).wait()

            @plsc.parallel_loop(0, EBLK // 16, unroll=8)
            def grp(g):
                si = bs[pl.ds(g * 16, 16)]
                di = bd[pl.ds(g * 16, 16)]
                av = plsc.load_gather(srow, [si])
                bv = plsc.load_gather(drow, [di])
                t = av + bv
                e = jnp.exp(jnp.maximum(t, 0.2 * t))
                plsc.addupdate_scatter(sacc, [di], e)
                for fr, ar in zip(feats, accs):
                    v = plsc.load_gather(fr, [si])
                    plsc.addupdate_scatter(ar, [di], v * e)

        issue(0, bufs[0])
        issue(1, bufs[1])

        def chunk_body(j, carry):
            for b in range(2):
                cc = 2 * j + b
                process(bufs[b])

                @pl.when(cc + 2 < nchunks)
                def _():
                    issue(cc + 2, bufs[b])

            return carry

        lax.fori_loop(0, nchunks // 2, chunk_body, 0)

        @plsc.parallel_loop(0, Nd // 16, unroll=4)
        def fin(i):
            sl = pl.ds(i * 16, 16)
            sv = sacc[sl] + 1e-16
            for a in accs:
                a[sl] = jnp.maximum(a[sl] / sv, 0.0)
        for r in range(4):
            pltpu.sync_copy(accs[r].at[pl.ds(0, Nd)],
                            outT.at[pl.ds((base + r) * Nd, Nd)])

    return edge_attn


def _edge_attn(srcT, ssrc, sdst, esrc, edst, n_dst):
    Ns = srcT.shape[1]
    E = esrc.shape[0]
    EP = ((E + 2 * EBLK - 1) // (2 * EBLK)) * (2 * EBLK)
    if EP != E:
        esrc = jnp.pad(esrc, (0, EP - E))
        edst = jnp.pad(edst, (0, EP - E), constant_values=n_dst)
    fn = _make_edge_attn(Ns, n_dst, EP)
    out = fn(srcT.reshape(-1), ssrc.reshape(-1), sdst.reshape(-1), esrc, edst)
    return out.reshape(C, n_dst)


def kernel(x_author, x_paper, edge_index_ap, edge_index_pa, edge_index_aa,
           W_a1, b_a1, W_p1, b_p1, asrc_ap1, adst_ap1, asrc_pa1, adst_pa1,
           asrc_aa1, adst_aa1, kW1, kb1, q1,
           W_a2, b_a2, W_p2, b_p2, asrc_ap2, adst_ap2, asrc_pa2, adst_pa2,
           asrc_aa2, adst_aa2, kW2, kb2, q2):
    na = x_author.shape[0]
    npp = x_paper.shape[0]

    A_a1 = jnp.concatenate(
        [_bd(asrc_ap1), _bd(adst_pa1), _bd(asrc_aa1), _bd(adst_aa1)], axis=0)
    A_p1 = jnp.concatenate([_bd(adst_ap1), _bd(asrc_pa1)], axis=0)
    zaT, zpT, Sa, Sp = _dense1(
        x_author, x_paper, W_a1, b_a1.reshape(C, 1), W_p1, b_p1.reshape(C, 1),
        A_a1, A_p1)

    p1T = _edge_attn(zaT, Sa[0:4], Sp[0:4],
                     edge_index_ap[0], edge_index_ap[1], npp)
    o1T = _edge_attn(zpT, Sp[4:8], Sa[4:8],
                     edge_index_pa[0], edge_index_pa[1], na)
    o2T = _edge_attn(zaT, Sa[8:12], Sa[12:16],
                     edge_index_aa[0], edge_index_aa[1], na)

    ks = _stats(o1T, o2T, kW1, kb1.reshape(C, 1), q1.reshape(C, 1))
    attn1 = jax.nn.softmax(ks.sum(axis=1) / na).reshape(1, 2)

    A_a2 = jnp.concatenate(
        [_bd(adst_pa2), _bd(asrc_aa2), _bd(adst_aa2),
         jnp.zeros((4, C), jnp.float32)], axis=0)
    A_p2 = jnp.concatenate(
        [_bd(asrc_pa2), jnp.zeros((4, C), jnp.float32)], axis=0)
    za2T, zp2T, Sa2, Sp2 = _dense2(
        o1T, o2T, attn1, p1T, W_a2, b_a2.reshape(C, 1),
        W_p2, b_p2.reshape(C, 1), A_a2, A_p2)

    o1T2 = _edge_attn(zp2T, Sp2[0:4], Sa2[0:4],
                      edge_index_pa[0], edge_index_pa[1], na)
    o2T2 = _edge_attn(za2T, Sa2[4:8], Sa2[8:12],
                      edge_index_aa[0], edge_index_aa[1], na)

    ks2 = _stats(o1T2, o2T2, kW2, kb2.reshape(C, 1), q2.reshape(C, 1))
    attn2 = jax.nn.softmax(ks2.sum(axis=1) / na).reshape(1, 2)
    return _final(o1T2, o2T2, attn2)


# speedup vs baseline: 1.4634x; 1.0032x over previous
import functools

import jax
import jax.numpy as jnp
from jax import lax
from jax.experimental import pallas as pl
from jax.experimental.pallas import tpu as pltpu
from jax.experimental.pallas import tpu_sc as plsc

H = 4
C = 128
NBLK = 2000
EBLK = 4000

_SC_MESH = plsc.VectorSubcoreMesh(core_axis_name="c", subcore_axis_name="s")
_SC_PARAMS = pltpu.CompilerParams(needs_layout_passes=False)


def _bd(a):
    return (jnp.eye(H, dtype=a.dtype)[:, :, None] * a[:, None, :]).reshape(H, C)


def _dense1_body(xa, xp, Wa, ba, Wp, bp, Aa, Ap, zaT, zpT, Sa, Sp):
    za = lax.dot_general(Wa[...], xa[...], (((0,), (1,)), ((), ())),
                         preferred_element_type=jnp.float32) + ba[...]
    zp = lax.dot_general(Wp[...], xp[...], (((0,), (1,)), ((), ())),
                         preferred_element_type=jnp.float32) + bp[...]
    zaT[...] = za
    zpT[...] = zp
    Sa[...] = lax.dot_general(Aa[...], za, (((1,), (0,)), ((), ())),
                              preferred_element_type=jnp.float32)
    Sp[...] = lax.dot_general(Ap[...], zp, (((1,), (0,)), ((), ())),
                              preferred_element_type=jnp.float32)


def _dense1(xa, xp, Wa, ba, Wp, bp, Aa, Ap):
    n = xa.shape[0]
    return pl.pallas_call(
        _dense1_body,
        out_shape=[
            jax.ShapeDtypeStruct((C, n), jnp.float32),
            jax.ShapeDtypeStruct((C, n), jnp.float32),
            jax.ShapeDtypeStruct((16, n), jnp.float32),
            jax.ShapeDtypeStruct((8, n), jnp.float32),
        ],
    )(xa, xp, Wa, ba, Wp, bp, Aa, Ap)


def _sem_attn(o1, o2, kW, kb, q):
    k1 = jnp.tanh(lax.dot_general(kW, o1, (((0,), (0,)), ((), ())),
                                  preferred_element_type=jnp.float32) + kb)
    k2 = jnp.tanh(lax.dot_general(kW, o2, (((0,), (0,)), ((), ())),
                                  preferred_element_type=jnp.float32) + kb)
    n = o1.shape[1]
    l1 = jnp.sum(k1 * q) / n
    l2 = jnp.sum(k2 * q) / n
    m = jnp.maximum(l1, l2)
    e1 = jnp.exp(l1 - m)
    e2 = jnp.exp(l2 - m)
    return e1 / (e1 + e2), e2 / (e1 + e2)


def _dense2_body(o1, o2, kW, kb, q, p1T, Wa, ba, Wp, bp, Aa, Ap,
                 za2, zp2, Sa2, Sp2):
    w0, w1 = _sem_attn(o1[...], o2[...], kW[...], kb[...], q[...])
    a1 = o1[...] * w0 + o2[...] * w1
    za = lax.dot_general(Wa[...], a1, (((0,), (0,)), ((), ())),
                         preferred_element_type=jnp.float32) + ba[...]
    zp = lax.dot_general(Wp[...], p1T[...], (((0,), (0,)), ((), ())),
                         preferred_element_type=jnp.float32) + bp[...]
    za2[...] = za
    zp2[...] = zp
    Sa2[...] = lax.dot_general(Aa[...], za, (((1,), (0,)), ((), ())),
                               preferred_element_type=jnp.float32)
    Sp2[...] = lax.dot_general(Ap[...], zp, (((1,), (0,)), ((), ())),
                               preferred_element_type=jnp.float32)


def _dense2(o1, o2, kW, kb, q, p1T, Wa, ba, Wp, bp, Aa, Ap):
    n = o1.shape[1]
    return pl.pallas_call(
        _dense2_body,
        out_shape=[
            jax.ShapeDtypeStruct((C, n), jnp.float32),
            jax.ShapeDtypeStruct((C, n), jnp.float32),
            jax.ShapeDtypeStruct((16, n), jnp.float32),
            jax.ShapeDtypeStruct((8, n), jnp.float32),
        ],
    )(o1, o2, kW, kb, q, p1T, Wa, ba, Wp, bp, Aa, Ap)


def _final_body(o1, o2, kW, kb, q, out):
    w0, w1 = _sem_attn(o1[...], o2[...], kW[...], kb[...], q[...])
    out[...] = lax.transpose(o1[...] * w0 + o2[...] * w1, (1, 0))


def _final(o1, o2, kW, kb, q):
    n = o1.shape[1]
    return pl.pallas_call(
        _final_body,
        out_shape=jax.ShapeDtypeStruct((n, C), jnp.float32),
    )(o1, o2, kW, kb, q)


@functools.lru_cache(maxsize=None)
def _make_edge_attn(Ns, Nd, EP):
    Nda = Nd + 16

    @functools.partial(
        pl.kernel,
        mesh=_SC_MESH,
        compiler_params=_SC_PARAMS,
        out_type=jax.ShapeDtypeStruct((C * Nd,), jnp.float32),
        scratch_types=(
            [pltpu.VMEM((Ns,), jnp.float32)] * 5
            + [pltpu.VMEM((Nda,), jnp.float32)] * 6
            + [pltpu.VMEM((EBLK,), jnp.int32)] * 4
            + [pltpu.SemaphoreType.DMA] * 2
        ),
    )
    def edge_attn(srcT, ssrc, sdst, esrc, edst, outT,
                  f0, f1, f2, f3, srow, drow, a0, a1, a2, a3, sacc,
                  bs0, bd0, bs1, bd1, sem0, sem1):
        wid = lax.axis_index("s") * 2 + lax.axis_index("c")
        h = wid // (32 // H)
        base = wid * (C // 32)
        feats = (f0, f1, f2, f3)
        accs = (a0, a1, a2, a3)
        for r in range(4):
            pltpu.sync_copy(srcT.at[pl.ds((base + r) * Ns, Ns)], feats[r])
        pltpu.sync_copy(ssrc.at[pl.ds(h * Ns, Ns)], srow)
        pltpu.sync_copy(sdst.at[pl.ds(h * Nd, Nd)], drow.at[pl.ds(0, Nd)])

        @plsc.parallel_loop(0, Nda // 16, unroll=4)
        def zero_acc(i):
            z = jnp.zeros((16,), jnp.float32)
            for a in accs:
                a[pl.ds(i * 16, 16)] = z
            sacc[pl.ds(i * 16, 16)] = z

        drow[pl.ds(Nd, 16)] = jnp.zeros((16,), jnp.float32)

        nchunks = EP // EBLK
        bufs = ((bs0, bd0, sem0), (bs1, bd1, sem1))

        def issue(cc, buf):
            bs, bd, sem = buf
            pltpu.async_copy(esrc.at[pl.ds(cc * EBLK, EBLK)], bs, sem)
            pltpu.async_copy(edst.at[pl.ds(cc * EBLK, EBLK)], bd, sem)

        def process(buf):
            bs, bd, sem = buf
            pltpu.make_async_copy(esrc.at[pl.ds(0, EBLK)], bs, sem).wait()
            pltpu.make_async_copy(edst.at[pl.ds(0, EBLK)], bd, sem).wait()

            @plsc.parallel_loop(0, EBLK // 16, unroll=8)
            def grp(g):
                si = bs[pl.ds(g * 16, 16)]
                di = bd[pl.ds(g * 16, 16)]
                av = plsc.load_gather(srow, [si])
                bv = plsc.load_gather(drow, [di])
                t = av + bv
                e = jnp.exp(jnp.maximum(t, 0.2 * t))
                plsc.addupdate_scatter(sacc, [di], e)
                for fr, ar in zip(feats, accs):
                    v = plsc.load_gather(fr, [si])
                    plsc.addupdate_scatter(ar, [di], v * e)

        issue(0, bufs[0])
        issue(1, bufs[1])

        def chunk_body(j, carry):
            for b in range(2):
                cc = 2 * j + b
                process(bufs[b])

                @pl.when(cc + 2 < nchunks)
                def _():
                    issue(cc + 2, bufs[b])

            return carry

        lax.fori_loop(0, nchunks // 2, chunk_body, 0)

        @plsc.parallel_loop(0, Nd // 16, unroll=4)
        def fin(i):
            sl = pl.ds(i * 16, 16)
            sv = sacc[sl] + 1e-16
            for a in accs:
                a[sl] = jnp.maximum(a[sl] / sv, 0.0)
        for r in range(4):
            pltpu.sync_copy(accs[r].at[pl.ds(0, Nd)],
                            outT.at[pl.ds((base + r) * Nd, Nd)])

    return edge_attn


def _edge_attn(srcT, ssrc, sdst, esrc, edst, n_dst):
    Ns = srcT.shape[1]
    E = esrc.shape[0]
    EP = ((E + 2 * EBLK - 1) // (2 * EBLK)) * (2 * EBLK)
    if EP != E:
        esrc = jnp.pad(esrc, (0, EP - E))
        edst = jnp.pad(edst, (0, EP - E), constant_values=n_dst)
    fn = _make_edge_attn(Ns, n_dst, EP)
    out = fn(srcT.reshape(-1), ssrc.reshape(-1), sdst.reshape(-1), esrc, edst)
    return out.reshape(C, n_dst)


def kernel(x_author, x_paper, edge_index_ap, edge_index_pa, edge_index_aa,
           W_a1, b_a1, W_p1, b_p1, asrc_ap1, adst_ap1, asrc_pa1, adst_pa1,
           asrc_aa1, adst_aa1, kW1, kb1, q1,
           W_a2, b_a2, W_p2, b_p2, asrc_ap2, adst_ap2, asrc_pa2, adst_pa2,
           asrc_aa2, adst_aa2, kW2, kb2, q2):
    na = x_author.shape[0]
    npp = x_paper.shape[0]

    A_a1 = jnp.concatenate(
        [_bd(asrc_ap1), _bd(adst_pa1), _bd(asrc_aa1), _bd(adst_aa1)], axis=0)
    A_p1 = jnp.concatenate([_bd(adst_ap1), _bd(asrc_pa1)], axis=0)
    zaT, zpT, Sa, Sp = _dense1(
        x_author, x_paper, W_a1, b_a1.reshape(C, 1), W_p1, b_p1.reshape(C, 1),
        A_a1, A_p1)

    p1T = _edge_attn(zaT, Sa[0:4], Sp[0:4],
                     edge_index_ap[0], edge_index_ap[1], npp)
    o1T = _edge_attn(zpT, Sp[4:8], Sa[4:8],
                     edge_index_pa[0], edge_index_pa[1], na)
    o2T = _edge_attn(zaT, Sa[8:12], Sa[12:16],
                     edge_index_aa[0], edge_index_aa[1], na)

    A_a2 = jnp.concatenate(
        [_bd(adst_pa2), _bd(asrc_aa2), _bd(adst_aa2),
         jnp.zeros((4, C), jnp.float32)], axis=0)
    A_p2 = jnp.concatenate(
        [_bd(asrc_pa2), jnp.zeros((4, C), jnp.float32)], axis=0)
    za2T, zp2T, Sa2, Sp2 = _dense2(
        o1T, o2T, kW1, kb1.reshape(C, 1), q1.reshape(C, 1), p1T,
        W_a2, b_a2.reshape(C, 1), W_p2, b_p2.reshape(C, 1), A_a2, A_p2)

    o1T2 = _edge_attn(zp2T, Sp2[0:4], Sa2[0:4],
                      edge_index_pa[0], edge_index_pa[1], na)
    o2T2 = _edge_attn(za2T, Sa2[4:8], Sa2[8:12],
                      edge_index_aa[0], edge_index_aa[1], na)

    return _final(o1T2, o2T2, kW2, kb2.reshape(C, 1), q2.reshape(C, 1))
